# Initial kernel scaffold; baseline (speedup 1.0000x reference)
import jax
import jax.numpy as jnp
from jax.experimental import pallas as pl


def kernel(enc_0, enc_1, edge_index, edge_weight, avg_edge_index, avg_edge_weight, W0, b0, a0, W1, b1):
    raise NotImplementedError("write your pallas kernel here")


# speedup vs baseline: 1.0173x; 1.0173x over previous
import jax
import jax.numpy as jnp
from jax.experimental import pallas as pl

_N = 10000
_D = 128
_GAMMA = (0.2, 0.2)
_BETA = 1.0


def _linear(x, W, b):
    BN = 2000

    def body(x_ref, w_ref, b_ref, o_ref):
        o_ref[...] = (
            jnp.dot(x_ref[...], w_ref[...], preferred_element_type=jnp.float32)
            + b_ref[...]
        )

    return pl.pallas_call(
        body,
        grid=(_N // BN,),
        in_specs=[
            pl.BlockSpec((BN, _D), lambda i: (i, 0)),
            pl.BlockSpec((_D, _D), lambda i: (0, 0)),
            pl.BlockSpec((_D,), lambda i: (0,)),
        ],
        out_specs=pl.BlockSpec((BN, _D), lambda i: (i, 0)),
        out_shape=jax.ShapeDtypeStruct((_N, _D), jnp.float32),
    )(x, W, b)


def _propagate_max(x, edge_index, edge_weight):
    src = edge_index[0]
    dst = edge_index[1]
    msg = x[src] * edge_weight[:, None]
    agg = jax.ops.segment_max(msg, dst, num_segments=_N)
    return jnp.where(jnp.isfinite(agg), agg, 0.0)


def _propagate_sum(x, edge_index, edge_weight):
    src = edge_index[0]
    dst = edge_index[1]
    msg = x[src] * edge_weight[:, None]
    return jax.ops.segment_sum(msg, dst, num_segments=_N)


def _deconv_wiener(x, W, b, edge_index, edge_weight, avg_edge_index, avg_edge_weight):
    h = _linear(x, W, b)
    out = h
    for g in _GAMMA:
        agg = _propagate_max(out, edge_index, edge_weight)
        avg_agg = _propagate_sum(out, avg_edge_index, avg_edge_weight)
        out = (1.0 + g) * out - agg + g * avg_agg
    return out


def kernel(enc_0, enc_1, edge_index, edge_weight, avg_edge_index, avg_edge_weight, W0, b0, a0, W1, b1):
    nk = jax.random.key(42)
    coef1 = jax.lax.stop_gradient(jnp.std(enc_1)) * _BETA
    noise1 = jax.random.normal(jax.random.fold_in(nk, 0), enc_1.shape, dtype=enc_1.dtype)
    dec0 = enc_1 + coef1 * noise1
    d = _deconv_wiener(dec0, W0, b0, edge_index, edge_weight, avg_edge_index, avg_edge_weight)
    d = jnp.where(d >= 0, d, a0 * d)
    coef2 = jax.lax.stop_gradient(jnp.std(enc_0)) * _BETA
    noise2 = jax.random.normal(jax.random.fold_in(nk, 1), enc_0.shape, dtype=enc_0.dtype)
    adv_enc = enc_0 + coef2 * noise2
    d1 = _deconv_wiener(adv_enc, W1, b1, edge_index, edge_weight, avg_edge_index, avg_edge_weight)
    d2 = _deconv_wiener(d, W1, b1, edge_index, edge_weight, avg_edge_index, avg_edge_weight)
    return d1 + d2
